# Initial kernel scaffold; baseline (speedup 1.0000x reference)
import jax
import jax.numpy as jnp
from jax.experimental import pallas as pl


def kernel(words, weight):
    raise NotImplementedError("write your pallas kernel here")


# speedup vs baseline: 1.3874x; 1.3874x over previous
import functools

import jax
import jax.numpy as jnp
from jax import lax
from jax.experimental import pallas as pl
from jax.experimental.pallas import tpu as pltpu
from jax.experimental.pallas import tpu_sc as plsc

VOCAB = 1000000
DIM = 64
P = 0.1
KEEP_THRESH = 0xE6666600
SCALE = float(1.0 / (1.0 - P))

ROWS128 = VOCAB * DIM // 128
MASK_BLOCK = 2000
N_LOOKUPS = 16384 * 50

NC, NS = 2, 16
NW = NC * NS
PER_W = N_LOOKUPS // NW
CHUNK = 128
N_CHUNKS = PER_W // CHUNK


def _threefry_key():
    def rotl(x, r):
        return ((x << r) | (x >> (32 - r))) & 0xFFFFFFFF

    def tf(k0, k1, c0, c1):
        ks = [k0, k1, k0 ^ k1 ^ 0x1BD11BDA]
        x0, x1 = (c0 + ks[0]) & 0xFFFFFFFF, (c1 + ks[1]) & 0xFFFFFFFF
        rots = [(13, 15, 26, 6), (17, 29, 16, 24)]
        for i in range(5):
            for r in rots[i % 2]:
                x0 = (x0 + x1) & 0xFFFFFFFF
                x1 = rotl(x1, r)
                x1 ^= x0
            x0 = (x0 + ks[(i + 1) % 3]) & 0xFFFFFFFF
            x1 = (x1 + ks[(i + 2) % 3] + i + 1) & 0xFFFFFFFF
        return x0, x1

    return tf(0, 0, 0, 123)


K0, K1 = _threefry_key()


def _mask_body(w_ref, o_ref):
    g = pl.program_id(0)
    w = w_ref[...]
    shp = (MASK_BLOCK, 128)
    row = lax.broadcasted_iota(jnp.uint32, shp, 0)
    col = lax.broadcasted_iota(jnp.uint32, shp, 1)
    base = (g * (MASK_BLOCK * 128)).astype(jnp.uint32)
    c1 = base + row * jnp.uint32(128) + col

    k0 = jnp.uint32(K0)
    k1 = jnp.uint32(K1)
    k2 = jnp.uint32(K0 ^ K1 ^ 0x1BD11BDA)
    ks = (k0, k1, k2)
    x0 = jnp.broadcast_to(k0, shp)
    x1 = c1 + k1
    rots = ((13, 15, 26, 6), (17, 29, 16, 24))
    for i in range(5):
        for r in rots[i % 2]:
            x0 = x0 + x1
            x1 = (x1 << jnp.uint32(r)) | (x1 >> jnp.uint32(32 - r))
            x1 = x1 ^ x0
        x0 = x0 + ks[(i + 1) % 3]
        x1 = x1 + ks[(i + 2) % 3] + jnp.uint32(i + 1)
    bits = x0 ^ x1
    keep = bits < jnp.uint32(KEEP_THRESH)
    o_ref[...] = jnp.where(keep, w * jnp.float32(SCALE), jnp.float32(0.0))


_mask_call = pl.pallas_call(
    _mask_body,
    grid=(ROWS128 // MASK_BLOCK,),
    in_specs=[pl.BlockSpec((MASK_BLOCK, 128), lambda i: (i, 0))],
    out_specs=pl.BlockSpec((MASK_BLOCK, 128), lambda i: (i, 0)),
    out_shape=jax.ShapeDtypeStruct((ROWS128, 128), jnp.float32),
)


def _gather_body(table, words_r, out, idx_v, buf, sem):
    wid = lax.axis_index("s") * NC + lax.axis_index("c")
    pltpu.sync_copy(words_r.at[wid], idx_v)
    base = wid * PER_W

    def step(j, carry):
        pltpu.async_copy(table.at[idx_v.at[j]], buf, sem).wait()
        pltpu.sync_copy(buf, out.at[pl.ds(base + j * CHUNK, CHUNK)])
        return carry

    lax.fori_loop(0, N_CHUNKS, step, 0)


@functools.cache
def _gather_call():
    return functools.partial(
        pl.kernel,
        out_type=jax.ShapeDtypeStruct((N_LOOKUPS, DIM), jnp.float32),
        mesh=plsc.VectorSubcoreMesh(core_axis_name="c", subcore_axis_name="s"),
        scratch_types=[
            pltpu.VMEM((N_CHUNKS, CHUNK), jnp.int32),
            pltpu.VMEM((CHUNK, DIM), jnp.float32),
            pltpu.SemaphoreType.DMA,
        ],
        compiler_params=pltpu.CompilerParams(use_tc_tiling_on_sc=False),
    )(_gather_body)


def kernel(words, weight):
    wf = weight.reshape(ROWS128, 128)
    masked = _mask_call(wf).reshape(VOCAB, DIM)
    words_r = words.reshape(NW, N_CHUNKS, CHUNK).astype(jnp.int32)
    out = _gather_call()(masked, words_r)
    return out.reshape(16384, 50, DIM)


# speedup vs baseline: 1.4721x; 1.0611x over previous
import functools

import jax
import jax.numpy as jnp
from jax import lax
from jax.experimental import pallas as pl
from jax.experimental.pallas import tpu as pltpu
from jax.experimental.pallas import tpu_sc as plsc

VOCAB = 1000000
DIM = 64
P = 0.1
KEEP_THRESH = 0xE6666600
SCALE = float(1.0 / (1.0 - P))

ROWS128 = VOCAB * DIM // 128
MASK_BLOCK = 2000
N_LOOKUPS = 16384 * 50

NC, NS = 2, 16
NW = NC * NS
PER_W = N_LOOKUPS // NW
CHUNK = 128
N_CHUNKS = PER_W // CHUNK


def _threefry_key():
    def rotl(x, r):
        return ((x << r) | (x >> (32 - r))) & 0xFFFFFFFF

    def tf(k0, k1, c0, c1):
        ks = [k0, k1, k0 ^ k1 ^ 0x1BD11BDA]
        x0, x1 = (c0 + ks[0]) & 0xFFFFFFFF, (c1 + ks[1]) & 0xFFFFFFFF
        rots = [(13, 15, 26, 6), (17, 29, 16, 24)]
        for i in range(5):
            for r in rots[i % 2]:
                x0 = (x0 + x1) & 0xFFFFFFFF
                x1 = rotl(x1, r)
                x1 ^= x0
            x0 = (x0 + ks[(i + 1) % 3]) & 0xFFFFFFFF
            x1 = (x1 + ks[(i + 2) % 3] + i + 1) & 0xFFFFFFFF
        return x0, x1

    return tf(0, 0, 0, 123)


K0, K1 = _threefry_key()


def _mask_body(w_ref, o_ref):
    g = pl.program_id(0)
    w = w_ref[...]
    shp = (MASK_BLOCK, 128)
    row = lax.broadcasted_iota(jnp.uint32, shp, 0)
    col = lax.broadcasted_iota(jnp.uint32, shp, 1)
    base = (g * (MASK_BLOCK * 128)).astype(jnp.uint32)
    c1 = base + row * jnp.uint32(128) + col

    k0 = jnp.uint32(K0)
    k1 = jnp.uint32(K1)
    k2 = jnp.uint32(K0 ^ K1 ^ 0x1BD11BDA)
    ks = (k0, k1, k2)
    x0 = jnp.broadcast_to(k0, shp)
    x1 = c1 + k1
    rots = ((13, 15, 26, 6), (17, 29, 16, 24))
    for i in range(5):
        for r in rots[i % 2]:
            x0 = x0 + x1
            x1 = (x1 << jnp.uint32(r)) | (x1 >> jnp.uint32(32 - r))
            x1 = x1 ^ x0
        x0 = x0 + ks[(i + 1) % 3]
        x1 = x1 + ks[(i + 2) % 3] + jnp.uint32(i + 1)
    bits = x0 ^ x1
    keep = bits < jnp.uint32(KEEP_THRESH)
    o_ref[...] = jnp.where(keep, w * jnp.float32(SCALE), jnp.float32(0.0))


_mask_call = pl.pallas_call(
    _mask_body,
    grid=(ROWS128 // MASK_BLOCK,),
    in_specs=[pl.BlockSpec((MASK_BLOCK, 128), lambda i: (i, 0))],
    out_specs=pl.BlockSpec((MASK_BLOCK, 128), lambda i: (i, 0)),
    out_shape=jax.ShapeDtypeStruct((ROWS128, 128), jnp.float32),
)


NBUF = 8


def _gather_body(table, words_r, out, idx_v, *rest):
    bufs = rest[:NBUF]
    gsems = rest[NBUF:2 * NBUF]
    wsems = rest[2 * NBUF:3 * NBUF]
    wid = lax.axis_index("s") * NC + lax.axis_index("c")
    pltpu.sync_copy(words_r.at[wid], idx_v)
    base = wid * PER_W

    def out_slice(j):
        return out.at[pl.ds(base + j * CHUNK, CHUNK)]

    for b in range(NBUF):
        pltpu.make_async_copy(bufs[b], out_slice(b), wsems[b]).start()

    def ring_round(r, carry):
        j0 = r * NBUF
        for b in range(NBUF):
            pltpu.make_async_copy(bufs[b], out_slice(j0 + b), wsems[b]).wait()
            pltpu.make_async_copy(
                table.at[idx_v.at[j0 + b]], bufs[b], gsems[b]).start()
        for b in range(NBUF):
            pltpu.make_async_copy(
                table.at[idx_v.at[j0 + b]], bufs[b], gsems[b]).wait()
            pltpu.make_async_copy(bufs[b], out_slice(j0 + b), wsems[b]).start()
        return carry

    lax.fori_loop(0, N_CHUNKS // NBUF, ring_round, 0)
    for b in range(NBUF):
        pltpu.make_async_copy(bufs[b], out_slice(b), wsems[b]).wait()


@functools.cache
def _gather_call():
    return functools.partial(
        pl.kernel,
        out_type=jax.ShapeDtypeStruct((N_LOOKUPS, DIM), jnp.float32),
        mesh=plsc.VectorSubcoreMesh(core_axis_name="c", subcore_axis_name="s"),
        scratch_types=(
            [pltpu.VMEM((N_CHUNKS, CHUNK), jnp.int32)]
            + [pltpu.VMEM((CHUNK, DIM), jnp.float32)] * NBUF
            + [pltpu.SemaphoreType.DMA] * (2 * NBUF)
        ),
        compiler_params=pltpu.CompilerParams(use_tc_tiling_on_sc=False),
    )(_gather_body)


def kernel(words, weight):
    wf = weight.reshape(ROWS128, 128)
    masked = _mask_call(wf).reshape(VOCAB, DIM)
    words_r = words.reshape(NW, N_CHUNKS, CHUNK).astype(jnp.int32)
    out = _gather_call()(masked, words_r)
    return out.reshape(16384, 50, DIM)
